# baseline (speedup 1.0000x reference)
import functools

import jax
import jax.numpy as jnp
import numpy as np
from jax.experimental import pallas as pl
from jax.experimental.pallas import tpu as pltpu

B = 5
N0 = 10000
POS = 16
HID = 64
HEADS = 2
KC = 10
TAU = 1.0
EPS = 1e-15


def _head_body(g_ref, w1_ref, b1_ref, w2_ref, b2_ref, out_ref):
    g = g_ref[...]
    h = jnp.maximum(jnp.dot(g, w1_ref[...]) + b1_ref[...], 0.0)
    out_ref[...] = jnp.dot(h, w2_ref[...]) + b2_ref[...]


def _head(g, params):
    return pl.pallas_call(
        _head_body,
        out_shape=jax.ShapeDtypeStruct((B, 2), jnp.float32),
    )(g, params['clf_W1'], params['clf_b1'][None, :],
      params['clf_W2'], params['clf_b2'][None, :])


def _gcn(x, src, dst, w, W, b):
    n = x.shape[0]
    xw = x @ W
    deg = jax.ops.segment_sum(w, dst, num_segments=n) + 1.0
    dinv = deg ** -0.5
    norm = dinv[src] * dinv[dst] * w
    out = jax.ops.segment_sum(norm[:, None] * xw[src], dst, num_segments=n)
    out = out + (1.0 / deg)[:, None] * xw
    return out + b


def _block(pb, x, src, dst, w, pos, n, first):
    z = x
    h = (x @ pb['proj_W']).reshape(-1, POS, HID)
    h = (h * pos[:, :, None]).sum(axis=1)
    h = _gcn(h, src, dst, w, pb['gcn_W'], pb['gcn_b'])
    mu = h.mean(axis=0)
    var = h.var(axis=0)
    h = (h - mu) / jnp.sqrt(var + 1e-5) * pb['bn_g'] + pb['bn_b']
    h = jax.nn.relu(h)
    if first:
        z = z @ pb['res_W'] + pb['res_b']
    h = h + z
    p = pb['pool_p']
    score = jnp.tanh((h @ p) / jnp.linalg.norm(p))
    k = int(np.ceil(0.5 * n))
    _, top_i = jax.lax.top_k(score.reshape(B, n), k)
    perm = (top_i + (jnp.arange(B, dtype=top_i.dtype) * n)[:, None]).reshape(-1)
    h_new = h[perm] * score[perm][:, None]
    old_n = B * n
    mapping = jnp.full((old_n,), -1, dtype=src.dtype).at[perm].set(
        jnp.arange(B * k, dtype=src.dtype))
    ns = mapping[src]
    nd = mapping[dst]
    valid = (ns >= 0) & (nd >= 0)
    w = w * valid.astype(w.dtype)
    src = jnp.where(valid, ns, 0)
    dst = jnp.where(valid, nd, 0)
    return h_new, src, dst, w, pos[perm], k


def kernel(x, edge_index, batch, params):
    src = edge_index[0].astype(jnp.int32)
    dst = edge_index[1].astype(jnp.int32)
    x_input = x.reshape(B, -1)
    pos_idx = jnp.tile(jnp.arange(N0), B)
    pos = jax.nn.softmax(params['pos_emb'][pos_idx], axis=-1)
    w = jnp.ones((src.shape[0],), jnp.float32)
    h = x
    n = N0
    for i, pb in enumerate(params['blocks']):
        h, src, dst, w, pos, n = _block(pb, h, src, dst, w, pos, n, i == 0)
    d2 = ((params['mem_k'].reshape(HEADS * KC, HID)[:, None, :] - h[None, :, :]) ** 2).sum(-1)
    d2 = (1.0 + d2 / TAU) ** (-(TAU + 1.0) / 2.0)
    d2 = d2.reshape(HEADS, KC, B, n).transpose(2, 3, 0, 1)
    S = d2 / d2.sum(axis=-1, keepdims=True)
    S = jnp.einsum('h,bnhk->bnk', params['mem_conv'], S)
    S = jax.nn.softmax(S, axis=-1)
    xd = h.reshape(B, n, HID)
    xp = jnp.einsum('bnk,bnd->bkd', S, xd) @ params['mem_lin']
    P = S ** 2 / S.sum(axis=1, keepdims=True)
    denom = P.sum(axis=2, keepdims=True)
    denom = jnp.where(S.sum(axis=2, keepdims=True) == 0.0, 1.0, denom)
    P = P / denom
    Pc = jnp.clip(P, EPS)
    kl = (Pc * (jnp.log(Pc) - jnp.log(jnp.clip(S, EPS)))).sum() / B
    g = xp.reshape(B, -1) @ params['fc1_W'] + params['fc1_b']
    g = g + x_input @ params['gres_W'] + params['gres_b']
    logits = _head(g, params)
    return logits, kl


# speedup vs baseline: 1.5140x; 1.5140x over previous
import functools

import jax
import jax.numpy as jnp
import numpy as np
from jax import lax
from jax.experimental import pallas as pl
from jax.experimental.pallas import tpu as pltpu
from jax.experimental.pallas import tpu_sc as plsc

B = 5
N0 = 10000
POS = 16
HID = 64
HEADS = 2
KC = 10
TAU = 1.0
EPS = 1e-15

LANES = 16
CHUNK = 128
NTILES = 16


def _npad(n):
    return ((n + 127) // 128) * 128


@functools.cache
def _spmm_fn(npad, e_pad):
    nch = e_pad // CHUNK // NTILES
    ept = nch * CHUNK
    wrpt = npad // NTILES
    zrpt = (npad + CHUNK) // NTILES
    mesh = plsc.VectorSubcoreMesh(core_axis_name="c", subcore_axis_name="s")

    @functools.partial(
        pl.kernel,
        out_type=jax.ShapeDtypeStruct((2 * npad, 32), jnp.float32),
        mesh=mesh,
        scratch_types=[
            pltpu.VMEM((CHUNK,), jnp.int32),
            pltpu.VMEM((CHUNK,), jnp.int32),
            pltpu.VMEM((CHUNK, 32), jnp.float32),
            pltpu.VMEM_SHARED((npad + CHUNK, 32), jnp.float32),
            pltpu.SemaphoreType.DMA,
        ],
        compiler_params=pltpu.CompilerParams(use_tc_tiling_on_sc=False),
    )
    def k(xs_hbm, src_hbm, route_hbm, zeros_hbm, out_hbm, sidx, ridx, rows, acc, sem):
        c = lax.axis_index("c")
        s = lax.axis_index("s")
        pltpu.sync_copy(zeros_hbm.at[pl.ds(0, zrpt)], acc.at[pl.ds(s * zrpt, zrpt)])
        plsc.subcore_barrier()
        cnp = c * npad
        e0 = s * ept

        def body(i, carry):
            base = e0 + i * CHUNK
            pltpu.sync_copy(src_hbm.at[pl.ds(base, CHUNK)], sidx)
            pltpu.sync_copy(route_hbm.at[pl.ds(base, CHUNK)], ridx)
            for j in range(CHUNK // LANES):
                sl = pl.ds(j * LANES, LANES)
                sidx[sl] = sidx[sl] + cnp
            pltpu.async_copy(xs_hbm.at[sidx], rows, sem).wait()
            pltpu.sync_copy(rows, acc.at[ridx], add=True)
            return carry

        lax.fori_loop(0, nch, body, 0)
        plsc.subcore_barrier()
        pltpu.sync_copy(acc.at[pl.ds(s * wrpt, wrpt)],
                        out_hbm.at[pl.ds(cnp + s * wrpt, wrpt)])

    return k


def _spmm(xs, src, route, n):
    npad = _npad(n)
    e_pad = src.shape[0]
    xs_pad = jnp.zeros((2 * npad, 32), jnp.float32)
    xs_pad = xs_pad.at[:n].set(xs[:, :32]).at[npad:npad + n].set(xs[:, 32:])
    zrpt = (npad + CHUNK) // NTILES
    zeros = jnp.zeros((zrpt, 32), jnp.float32)
    out = _spmm_fn(npad, e_pad)(xs_pad, src, route, zeros)
    return jnp.concatenate([out[:n], out[npad:npad + n]], axis=1)


def _head_body(g_ref, w1_ref, b1_ref, w2_ref, b2_ref, out_ref):
    g = g_ref[...]
    h = jnp.maximum(jnp.dot(g, w1_ref[...]) + b1_ref[...], 0.0)
    out_ref[...] = jnp.dot(h, w2_ref[...]) + b2_ref[...]


def _head(g, params):
    return pl.pallas_call(
        _head_body,
        out_shape=jax.ShapeDtypeStruct((B, 2), jnp.float32),
    )(g, params['clf_W1'], params['clf_b1'][None, :],
      params['clf_W2'], params['clf_b2'][None, :])


def _gcn(x, src, dst, w, W, b):
    n = x.shape[0]
    xw = x @ W
    deg = jax.ops.segment_sum(w, dst, num_segments=n) + 1.0
    dinv = deg ** -0.5
    xs = dinv[:, None] * xw
    route = jnp.where(w > 0, dst, _npad(n)).astype(jnp.int32)
    msg = _spmm(xs, src, route, n)
    out = dinv[:, None] * msg + (1.0 / deg)[:, None] * xw
    return out + b


def _block(pb, x, src, dst, w, pos, n, first):
    z = x
    h = (x @ pb['proj_W']).reshape(-1, POS, HID)
    h = (h * pos[:, :, None]).sum(axis=1)
    h = _gcn(h, src, dst, w, pb['gcn_W'], pb['gcn_b'])
    mu = h.mean(axis=0)
    var = h.var(axis=0)
    h = (h - mu) / jnp.sqrt(var + 1e-5) * pb['bn_g'] + pb['bn_b']
    h = jax.nn.relu(h)
    if first:
        z = z @ pb['res_W'] + pb['res_b']
    h = h + z
    p = pb['pool_p']
    score = jnp.tanh((h @ p) / jnp.linalg.norm(p))
    k = int(np.ceil(0.5 * n))
    _, top_i = jax.lax.top_k(score.reshape(B, n), k)
    perm = (top_i + (jnp.arange(B, dtype=top_i.dtype) * n)[:, None]).reshape(-1)
    h_new = h[perm] * score[perm][:, None]
    old_n = B * n
    mapping = jnp.full((old_n,), -1, dtype=src.dtype).at[perm].set(
        jnp.arange(B * k, dtype=src.dtype))
    ns = mapping[src]
    nd = mapping[dst]
    valid = (ns >= 0) & (nd >= 0)
    w = w * valid.astype(w.dtype)
    src = jnp.where(valid, ns, 0)
    dst = jnp.where(valid, nd, 0)
    return h_new, src, dst, w, pos[perm], k


def kernel(x, edge_index, batch, params):
    E = edge_index.shape[1]
    e_pad = ((E + CHUNK * NTILES - 1) // (CHUNK * NTILES)) * (CHUNK * NTILES)
    src = jnp.zeros((e_pad,), jnp.int32).at[:E].set(edge_index[0].astype(jnp.int32))
    dst = jnp.zeros((e_pad,), jnp.int32).at[:E].set(edge_index[1].astype(jnp.int32))
    w = jnp.zeros((e_pad,), jnp.float32).at[:E].set(1.0)
    x_input = x.reshape(B, -1)
    pos_idx = jnp.tile(jnp.arange(N0), B)
    pos = jax.nn.softmax(params['pos_emb'][pos_idx], axis=-1)
    h = x
    n = N0
    for i, pb in enumerate(params['blocks']):
        h, src, dst, w, pos, n = _block(pb, h, src, dst, w, pos, n, i == 0)
    d2 = ((params['mem_k'].reshape(HEADS * KC, HID)[:, None, :] - h[None, :, :]) ** 2).sum(-1)
    d2 = (1.0 + d2 / TAU) ** (-(TAU + 1.0) / 2.0)
    d2 = d2.reshape(HEADS, KC, B, n).transpose(2, 3, 0, 1)
    S = d2 / d2.sum(axis=-1, keepdims=True)
    S = jnp.einsum('h,bnhk->bnk', params['mem_conv'], S)
    S = jax.nn.softmax(S, axis=-1)
    xd = h.reshape(B, n, HID)
    xp = jnp.einsum('bnk,bnd->bkd', S, xd) @ params['mem_lin']
    P = S ** 2 / S.sum(axis=1, keepdims=True)
    denom = P.sum(axis=2, keepdims=True)
    denom = jnp.where(S.sum(axis=2, keepdims=True) == 0.0, 1.0, denom)
    P = P / denom
    Pc = jnp.clip(P, EPS)
    kl = (Pc * (jnp.log(Pc) - jnp.log(jnp.clip(S, EPS)))).sum() / B
    g = xp.reshape(B, -1) @ params['fc1_W'] + params['fc1_b']
    g = g + x_input @ params['gres_W'] + params['gres_b']
    logits = _head(g, params)
    return logits, kl
